# baseline (speedup 1.0000x reference)
import functools

import jax
import jax.numpy as jnp
from jax import lax
from jax.experimental import pallas as pl
from jax.experimental.pallas import tpu as pltpu
from jax.experimental.pallas import tpu_sc as plsc

N = 10000
E = 320000
P = 65536
D = 128
NC = 2
NS = 16
NW = NC * NS
C = 128
CPW = 80
CPW0 = 110
CPW1 = 50
EPAD = NW * CPW * C + 2 * C
SROWS = 10240
RPT = SROWS // NS
PPW = P // NW
PCW = PPW // C
PPAD = P + 2 * C
BR = 256


def _histogram(dst_ref, sbuf, deg_v, lane_iota):
    for kk in range(8):
        d16 = dst_ref[pl.ds(kk * 16, 16)]
        sk, _ = plsc.sort_key_val(d16, d16)
        sbuf[pl.ds(1, 16)] = sk
        prev = sbuf[pl.ds(0, 16)]
        nxt = sbuf[pl.ds(2, 16)]
        run_start = plsc.cummax(jnp.where(sk != prev, lane_iota, -1))
        cnt = (lane_iota - run_start + 1).astype(jnp.float32)
        plsc.addupdate_scatter(deg_v, [sk], cnt, mask=sk != nxt)


def _aggregate_body(x_h, srcf_h, dstf_h, zrows_h, zdeg_h, part_h, degf_h,
                    src_c0, src_c1, dst_c0, dst_c1, rows0, rows1,
                    deg_v, sbuf, agg_sh, sem_i, sem_g):
    cid = lax.axis_index("c")
    sid = lax.axis_index("s")
    wid = cid * NS + sid
    lane_iota = lax.iota(jnp.int32, 16)
    src_c = (src_c0, src_c1)
    dst_c = (dst_c0, dst_c1)
    rows = (rows0, rows1)
    cpw = jnp.where(cid == 0, CPW0, CPW1)
    cbase = cid * (NS * CPW0) + sid * cpw

    def base(j):
        return (cbase + j) * C

    def drain_idx():
        pltpu.make_async_copy(srcf_h.at[pl.ds(0, C)], src_c0, sem_i).wait()
        pltpu.make_async_copy(srcf_h.at[pl.ds(0, C)], dst_c0, sem_i).wait()

    def fire_idx(j, k):
        pltpu.async_copy(srcf_h.at[pl.ds(base(j), C)], src_c[k], sem_i)
        pltpu.async_copy(dstf_h.at[pl.ds(base(j), C)], dst_c[k], sem_i)

    pltpu.sync_copy(zrows_h, agg_sh.at[pl.ds(sid * RPT, RPT)])
    pltpu.sync_copy(zdeg_h, deg_v)
    sbuf[pl.ds(0, 16)] = jnp.full((16,), -1, jnp.int32)
    sbuf[pl.ds(16, 16)] = jnp.full((16,), -2, jnp.int32)
    plsc.subcore_barrier()

    fire_idx(0, 0)
    fire_idx(1, 1)
    drain_idx()
    pltpu.async_copy(x_h.at[src_c0], rows0, sem_g)

    @pl.loop(0, cpw, step=2)
    def _(jj):
        for b in range(2):
            j = jj + b
            k, o = b, 1 - b
            pltpu.make_async_copy(x_h.at[pl.ds(0, C)], rows[k],
                                  sem_g).wait()
            drain_idx()
            pltpu.async_copy(x_h.at[src_c[o]], rows[o], sem_g)
            pltpu.sync_copy(rows[k], agg_sh.at[dst_c[k]], add=True)
            _histogram(dst_c[k], sbuf, deg_v, lane_iota)
            fire_idx(j + 2, k)

    pltpu.make_async_copy(x_h.at[pl.ds(0, C)], rows0, sem_g).wait()
    drain_idx()

    plsc.subcore_barrier()
    pltpu.sync_copy(agg_sh.at[pl.ds(sid * RPT, RPT)],
                    part_h.at[cid, pl.ds(sid * RPT, RPT)])
    pltpu.sync_copy(deg_v, degf_h.at[pl.ds(wid * SROWS, SROWS)])


def _encode_body(part_ref, degp_ref, w_ref, h_ref):
    agg = part_ref[0] + part_ref[1]
    deg = jnp.maximum(jnp.sum(degp_ref[...], axis=0), 1.0)
    h = jnp.dot(agg / deg[:, None], w_ref[...],
                preferred_element_type=jnp.float32)
    h_ref[...] = jnp.maximum(h, 0.0)


def _predict_body(h_h, aif_h, bif_h, pred_h,
                  ai_c0, ai_c1, bi_c0, bi_c1, ar0, ar1, br0, br1, ob,
                  sem_i, sem_g):
    cid = lax.axis_index("c")
    sid = lax.axis_index("s")
    wid = cid * NS + sid
    lane_iota = lax.iota(jnp.int32, 16)
    ai_c = (ai_c0, ai_c1)
    bi_c = (bi_c0, bi_c1)
    ar = (ar0, ar1)
    br = (br0, br1)

    def base(j):
        return wid * PPW + j * C

    def drain_idx():
        pltpu.make_async_copy(aif_h.at[pl.ds(0, C)], ai_c0, sem_i).wait()
        pltpu.make_async_copy(aif_h.at[pl.ds(0, C)], bi_c0, sem_i).wait()

    def fire_idx(j, k):
        pltpu.async_copy(aif_h.at[pl.ds(base(j), C)], ai_c[k], sem_i)
        pltpu.async_copy(bif_h.at[pl.ds(base(j), C)], bi_c[k], sem_i)

    def drain_rows(k):
        pltpu.make_async_copy(h_h.at[pl.ds(0, C)], ar[k], sem_g).wait()
        pltpu.make_async_copy(h_h.at[pl.ds(0, C)], br[k], sem_g).wait()

    def fire_rows(k):
        pltpu.async_copy(h_h.at[ai_c[k]], ar[k], sem_g)
        pltpu.async_copy(h_h.at[bi_c[k]], br[k], sem_g)

    fire_idx(0, 0)
    fire_idx(1, 1)
    drain_idx()
    fire_rows(0)

    @pl.loop(0, PCW, step=2)
    def _(jj):
        for b in range(2):
            j = jj + b
            k, o = b, 1 - b
            drain_rows(k)
            drain_idx()
            fire_rows(o)
            fire_idx(j + 2, k)

            def ebody(e, out16):
                a_e = ar[k].at[e]
                b_e = br[k].at[e]
                acc = a_e[pl.ds(0, 16)] * b_e[pl.ds(0, 16)]
                for t in range(1, 8):
                    acc = acc + (a_e[pl.ds(t * 16, 16)]
                                 * b_e[pl.ds(t * 16, 16)])
                s = jnp.sum(acc)
                lane = e % 16
                out16 = jnp.where(lane_iota == lane, s, out16)

                @pl.when(lane == 15)
                def _store():
                    ob[pl.ds(j * C + e - 15, 16)] = out16

                return out16

            lax.fori_loop(0, C, ebody, jnp.zeros((16,), jnp.float32))

    drain_rows(0)
    drain_idx()
    pltpu.sync_copy(ob, pred_h.at[pl.ds(wid * PPW, PPW)])


def kernel(x, edge_index, edge_label_index, W):
    x = x.astype(jnp.float32)
    src = edge_index[0].astype(jnp.int32)
    dst = edge_index[1].astype(jnp.int32)
    pad = EPAD - E
    srcf = jnp.concatenate([src, jnp.zeros((pad,), jnp.int32)])
    dstf = jnp.concatenate([dst, jnp.full((pad,), SROWS - 1, jnp.int32)])
    zrows = jnp.zeros((RPT, D), jnp.float32)
    zdeg = jnp.zeros((SROWS,), jnp.float32)
    lpad = jnp.zeros((PPAD - P,), jnp.int32)
    aif = jnp.concatenate([edge_label_index[0].astype(jnp.int32), lpad])
    bif = jnp.concatenate([edge_label_index[1].astype(jnp.int32), lpad])

    mesh = plsc.VectorSubcoreMesh(core_axis_name="c", subcore_axis_name="s")
    sc_params = pltpu.CompilerParams(needs_layout_passes=False)

    aggregate = functools.partial(
        pl.kernel,
        out_type=(
            jax.ShapeDtypeStruct((NC, SROWS, D), jnp.float32),
            jax.ShapeDtypeStruct((NW * SROWS,), jnp.float32),
        ),
        mesh=mesh,
        scratch_types=[
            pltpu.VMEM((C,), jnp.int32),
            pltpu.VMEM((C,), jnp.int32),
            pltpu.VMEM((C,), jnp.int32),
            pltpu.VMEM((C,), jnp.int32),
            pltpu.VMEM((C, D), jnp.float32),
            pltpu.VMEM((C, D), jnp.float32),
            pltpu.VMEM((SROWS,), jnp.float32),
            pltpu.VMEM((32,), jnp.int32),
            pltpu.VMEM_SHARED((SROWS, D), jnp.float32),
            pltpu.SemaphoreType.DMA,
            pltpu.SemaphoreType.DMA,
        ],
        compiler_params=sc_params,
    )(_aggregate_body)
    part, degf = aggregate(x, srcf, dstf, zrows, zdeg)
    degp = degf.reshape(NW, SROWS)

    h = pl.pallas_call(
        _encode_body,
        grid=(SROWS // BR,),
        in_specs=[
            pl.BlockSpec((NC, BR, D), lambda i: (0, i, 0)),
            pl.BlockSpec((NW, BR), lambda i: (0, i)),
            pl.BlockSpec((D, D), lambda i: (0, 0)),
        ],
        out_specs=pl.BlockSpec((BR, D), lambda i: (i, 0)),
        out_shape=jax.ShapeDtypeStruct((SROWS, D), jnp.float32),
    )(part, degp, W.astype(jnp.float32))

    predict = functools.partial(
        pl.kernel,
        out_type=jax.ShapeDtypeStruct((P,), jnp.float32),
        mesh=mesh,
        scratch_types=[
            pltpu.VMEM((C,), jnp.int32),
            pltpu.VMEM((C,), jnp.int32),
            pltpu.VMEM((C,), jnp.int32),
            pltpu.VMEM((C,), jnp.int32),
            pltpu.VMEM((C, D), jnp.float32),
            pltpu.VMEM((C, D), jnp.float32),
            pltpu.VMEM((C, D), jnp.float32),
            pltpu.VMEM((C, D), jnp.float32),
            pltpu.VMEM((PPW,), jnp.float32),
            pltpu.SemaphoreType.DMA,
            pltpu.SemaphoreType.DMA,
        ],
        compiler_params=sc_params,
    )(_predict_body)
    return predict(h, aif, bif)


# speedup vs baseline: 1.1442x; 1.1442x over previous
import functools

import jax
import jax.numpy as jnp
from jax import lax
from jax.experimental import pallas as pl
from jax.experimental.pallas import tpu as pltpu
from jax.experimental.pallas import tpu_sc as plsc

N = 10000
E = 320000
P = 65536
D = 128
NC = 2
NS = 16
NW = NC * NS
C = 128
CPW = 80
CPW0 = 138
CPW1 = 22
EPAD = NW * CPW * C + 2 * C
SROWS = 10240
RPT = SROWS // NS
PPW = P // NW
PCW = PPW // C
PPAD = P + 2 * C
BR = 256


def _histogram(dst_ref, sbuf, deg_v, lane_iota):
    for kk in range(8):
        d16 = dst_ref[pl.ds(kk * 16, 16)]
        sk, _ = plsc.sort_key_val(d16, d16)
        sbuf[pl.ds(1, 16)] = sk
        prev = sbuf[pl.ds(0, 16)]
        nxt = sbuf[pl.ds(2, 16)]
        run_start = plsc.cummax(jnp.where(sk != prev, lane_iota, -1))
        cnt = (lane_iota - run_start + 1).astype(jnp.float32)
        plsc.addupdate_scatter(deg_v, [sk], cnt, mask=sk != nxt)


def _aggregate_body(x_h, srcf_h, dstf_h, zrows_h, zdeg_h, part_h, degf_h,
                    src_c0, src_c1, dst_c0, dst_c1, rows0, rows1,
                    deg_v, sbuf, agg_sh, sem_i, sem_g):
    cid = lax.axis_index("c")
    sid = lax.axis_index("s")
    wid = cid * NS + sid
    lane_iota = lax.iota(jnp.int32, 16)
    src_c = (src_c0, src_c1)
    dst_c = (dst_c0, dst_c1)
    rows = (rows0, rows1)
    cpw = jnp.where(cid == 0, CPW0, CPW1)
    cbase = cid * (NS * CPW0) + sid * cpw

    def base(j):
        return (cbase + j) * C

    def drain_idx():
        pltpu.make_async_copy(srcf_h.at[pl.ds(0, C)], src_c0, sem_i).wait()
        pltpu.make_async_copy(srcf_h.at[pl.ds(0, C)], dst_c0, sem_i).wait()

    def fire_idx(j, k):
        pltpu.async_copy(srcf_h.at[pl.ds(base(j), C)], src_c[k], sem_i)
        pltpu.async_copy(dstf_h.at[pl.ds(base(j), C)], dst_c[k], sem_i)

    pltpu.sync_copy(zrows_h, agg_sh.at[pl.ds(sid * RPT, RPT)])
    pltpu.sync_copy(zdeg_h, deg_v)
    sbuf[pl.ds(0, 16)] = jnp.full((16,), -1, jnp.int32)
    sbuf[pl.ds(16, 16)] = jnp.full((16,), -2, jnp.int32)
    plsc.subcore_barrier()

    fire_idx(0, 0)
    fire_idx(1, 1)
    drain_idx()
    pltpu.async_copy(x_h.at[src_c0], rows0, sem_g)

    @pl.loop(0, cpw, step=2)
    def _(jj):
        for b in range(2):
            j = jj + b
            k, o = b, 1 - b
            pltpu.make_async_copy(x_h.at[pl.ds(0, C)], rows[k],
                                  sem_g).wait()
            drain_idx()
            pltpu.async_copy(x_h.at[src_c[o]], rows[o], sem_g)
            pltpu.sync_copy(rows[k], agg_sh.at[dst_c[k]], add=True)
            _histogram(dst_c[k], sbuf, deg_v, lane_iota)
            fire_idx(j + 2, k)

    pltpu.make_async_copy(x_h.at[pl.ds(0, C)], rows0, sem_g).wait()
    drain_idx()

    plsc.subcore_barrier()
    pltpu.sync_copy(agg_sh.at[pl.ds(sid * RPT, RPT)],
                    part_h.at[cid, pl.ds(sid * RPT, RPT)])
    pltpu.sync_copy(deg_v, degf_h.at[pl.ds(wid * SROWS, SROWS)])


def _encode_body(part_ref, degp_ref, w_ref, h_ref):
    agg = part_ref[0] + part_ref[1]
    deg = jnp.maximum(jnp.sum(degp_ref[...], axis=0), 1.0)
    h = jnp.dot(agg / deg[:, None], w_ref[...],
                preferred_element_type=jnp.float32)
    h_ref[...] = jnp.maximum(h, 0.0)


def _predict_body(h_h, aif_h, bif_h, pred_h,
                  ai_c0, ai_c1, bi_c0, bi_c1, ar0, ar1, br0, br1, ob,
                  sem_i, sem_g):
    cid = lax.axis_index("c")
    sid = lax.axis_index("s")
    wid = cid * NS + sid
    lane_iota = lax.iota(jnp.int32, 16)
    ai_c = (ai_c0, ai_c1)
    bi_c = (bi_c0, bi_c1)
    ar = (ar0, ar1)
    br = (br0, br1)

    def base(j):
        return wid * PPW + j * C

    def drain_idx():
        pltpu.make_async_copy(aif_h.at[pl.ds(0, C)], ai_c0, sem_i).wait()
        pltpu.make_async_copy(aif_h.at[pl.ds(0, C)], bi_c0, sem_i).wait()

    def fire_idx(j, k):
        pltpu.async_copy(aif_h.at[pl.ds(base(j), C)], ai_c[k], sem_i)
        pltpu.async_copy(bif_h.at[pl.ds(base(j), C)], bi_c[k], sem_i)

    def drain_rows(k):
        pltpu.make_async_copy(h_h.at[pl.ds(0, C)], ar[k], sem_g).wait()
        pltpu.make_async_copy(h_h.at[pl.ds(0, C)], br[k], sem_g).wait()

    def fire_rows(k):
        pltpu.async_copy(h_h.at[ai_c[k]], ar[k], sem_g)
        pltpu.async_copy(h_h.at[bi_c[k]], br[k], sem_g)

    fire_idx(0, 0)
    fire_idx(1, 1)
    drain_idx()
    fire_rows(0)

    @pl.loop(0, PCW, step=2)
    def _(jj):
        for b in range(2):
            j = jj + b
            k, o = b, 1 - b
            drain_rows(k)
            drain_idx()
            fire_rows(o)
            fire_idx(j + 2, k)

            def ebody(e, out16):
                a_e = ar[k].at[e]
                b_e = br[k].at[e]
                acc = a_e[pl.ds(0, 16)] * b_e[pl.ds(0, 16)]
                for t in range(1, 8):
                    acc = acc + (a_e[pl.ds(t * 16, 16)]
                                 * b_e[pl.ds(t * 16, 16)])
                s = jnp.sum(acc)
                lane = e % 16
                out16 = jnp.where(lane_iota == lane, s, out16)

                @pl.when(lane == 15)
                def _store():
                    ob[pl.ds(j * C + e - 15, 16)] = out16

                return out16

            lax.fori_loop(0, C, ebody, jnp.zeros((16,), jnp.float32))

    drain_rows(0)
    drain_idx()
    pltpu.sync_copy(ob, pred_h.at[pl.ds(wid * PPW, PPW)])


def kernel(x, edge_index, edge_label_index, W):
    x = x.astype(jnp.float32)
    src = edge_index[0].astype(jnp.int32)
    dst = edge_index[1].astype(jnp.int32)
    pad = EPAD - E
    srcf = jnp.concatenate([src, jnp.zeros((pad,), jnp.int32)])
    dstf = jnp.concatenate([dst, jnp.full((pad,), SROWS - 1, jnp.int32)])
    zrows = jnp.zeros((RPT, D), jnp.float32)
    zdeg = jnp.zeros((SROWS,), jnp.float32)
    lpad = jnp.zeros((PPAD - P,), jnp.int32)
    aif = jnp.concatenate([edge_label_index[0].astype(jnp.int32), lpad])
    bif = jnp.concatenate([edge_label_index[1].astype(jnp.int32), lpad])

    mesh = plsc.VectorSubcoreMesh(core_axis_name="c", subcore_axis_name="s")
    sc_params = pltpu.CompilerParams(needs_layout_passes=False)

    aggregate = functools.partial(
        pl.kernel,
        out_type=(
            jax.ShapeDtypeStruct((NC, SROWS, D), jnp.float32),
            jax.ShapeDtypeStruct((NW * SROWS,), jnp.float32),
        ),
        mesh=mesh,
        scratch_types=[
            pltpu.VMEM((C,), jnp.int32),
            pltpu.VMEM((C,), jnp.int32),
            pltpu.VMEM((C,), jnp.int32),
            pltpu.VMEM((C,), jnp.int32),
            pltpu.VMEM((C, D), jnp.float32),
            pltpu.VMEM((C, D), jnp.float32),
            pltpu.VMEM((SROWS,), jnp.float32),
            pltpu.VMEM((32,), jnp.int32),
            pltpu.VMEM_SHARED((SROWS, D), jnp.float32),
            pltpu.SemaphoreType.DMA,
            pltpu.SemaphoreType.DMA,
        ],
        compiler_params=sc_params,
    )(_aggregate_body)
    part, degf = aggregate(x, srcf, dstf, zrows, zdeg)
    degp = degf.reshape(NW, SROWS)

    h = pl.pallas_call(
        _encode_body,
        grid=(SROWS // BR,),
        in_specs=[
            pl.BlockSpec((NC, BR, D), lambda i: (0, i, 0)),
            pl.BlockSpec((NW, BR), lambda i: (0, i)),
            pl.BlockSpec((D, D), lambda i: (0, 0)),
        ],
        out_specs=pl.BlockSpec((BR, D), lambda i: (i, 0)),
        out_shape=jax.ShapeDtypeStruct((SROWS, D), jnp.float32),
    )(part, degp, W.astype(jnp.float32))

    predict = functools.partial(
        pl.kernel,
        out_type=jax.ShapeDtypeStruct((P,), jnp.float32),
        mesh=mesh,
        scratch_types=[
            pltpu.VMEM((C,), jnp.int32),
            pltpu.VMEM((C,), jnp.int32),
            pltpu.VMEM((C,), jnp.int32),
            pltpu.VMEM((C,), jnp.int32),
            pltpu.VMEM((C, D), jnp.float32),
            pltpu.VMEM((C, D), jnp.float32),
            pltpu.VMEM((C, D), jnp.float32),
            pltpu.VMEM((C, D), jnp.float32),
            pltpu.VMEM((PPW,), jnp.float32),
            pltpu.SemaphoreType.DMA,
            pltpu.SemaphoreType.DMA,
        ],
        compiler_params=sc_params,
    )(_predict_body)
    return predict(h, aif, bif)


# speedup vs baseline: 1.2682x; 1.1084x over previous
import functools

import jax
import jax.numpy as jnp
from jax import lax
from jax.experimental import pallas as pl
from jax.experimental.pallas import tpu as pltpu
from jax.experimental.pallas import tpu_sc as plsc

N = 10000
E = 320000
P = 65536
D = 128
NC = 2
NS = 16
NW = NC * NS
C = 128
CPW = 80
CPW0 = 146
CPW1 = 14
EPAD = NW * CPW * C + 2 * C
SROWS = 10240
RPT = SROWS // NS
PPW = P // NW
PCW = PPW // C
PPAD = P + 2 * C
BR = 256


def _histogram(dst_ref, sbuf, deg_v, lane_iota):
    for kk in range(8):
        d16 = dst_ref[pl.ds(kk * 16, 16)]
        sk, _ = plsc.sort_key_val(d16, d16)
        sbuf[pl.ds(1, 16)] = sk
        prev = sbuf[pl.ds(0, 16)]
        nxt = sbuf[pl.ds(2, 16)]
        run_start = plsc.cummax(jnp.where(sk != prev, lane_iota, -1))
        cnt = (lane_iota - run_start + 1).astype(jnp.float32)
        plsc.addupdate_scatter(deg_v, [sk], cnt, mask=sk != nxt)


def _aggregate_body(x_h, srcf_h, dstf_h, zrows_h, zdeg_h, part_h, degf_h,
                    src_c0, src_c1, dst_c0, dst_c1, rows0, rows1,
                    deg_v, sbuf, agg_sh, sem_i, sem_g):
    cid = lax.axis_index("c")
    sid = lax.axis_index("s")
    wid = cid * NS + sid
    lane_iota = lax.iota(jnp.int32, 16)
    src_c = (src_c0, src_c1)
    dst_c = (dst_c0, dst_c1)
    rows = (rows0, rows1)
    cpw = jnp.where(cid == 0, CPW0, CPW1)
    cbase = cid * (NS * CPW0) + sid * cpw

    def base(j):
        return (cbase + j) * C

    def drain_idx():
        pltpu.make_async_copy(srcf_h.at[pl.ds(0, C)], src_c0, sem_i).wait()
        pltpu.make_async_copy(srcf_h.at[pl.ds(0, C)], dst_c0, sem_i).wait()

    def fire_idx(j, k):
        pltpu.async_copy(srcf_h.at[pl.ds(base(j), C)], src_c[k], sem_i)
        pltpu.async_copy(dstf_h.at[pl.ds(base(j), C)], dst_c[k], sem_i)

    pltpu.sync_copy(zrows_h, agg_sh.at[pl.ds(sid * RPT, RPT)])
    pltpu.sync_copy(zdeg_h, deg_v)
    sbuf[pl.ds(0, 16)] = jnp.full((16,), -1, jnp.int32)
    sbuf[pl.ds(16, 16)] = jnp.full((16,), -2, jnp.int32)
    plsc.subcore_barrier()

    fire_idx(0, 0)
    fire_idx(1, 1)
    drain_idx()
    pltpu.async_copy(x_h.at[src_c0], rows0, sem_g)

    @pl.loop(0, cpw, step=2)
    def _(jj):
        for b in range(2):
            j = jj + b
            k, o = b, 1 - b
            pltpu.make_async_copy(x_h.at[pl.ds(0, C)], rows[k],
                                  sem_g).wait()
            drain_idx()
            pltpu.async_copy(x_h.at[src_c[o]], rows[o], sem_g)
            pltpu.sync_copy(rows[k], agg_sh.at[dst_c[k]], add=True)
            _histogram(dst_c[k], sbuf, deg_v, lane_iota)
            fire_idx(j + 2, k)

    pltpu.make_async_copy(x_h.at[pl.ds(0, C)], rows0, sem_g).wait()
    drain_idx()

    plsc.subcore_barrier()
    pltpu.sync_copy(agg_sh.at[pl.ds(sid * RPT, RPT)],
                    part_h.at[cid, pl.ds(sid * RPT, RPT)])
    pltpu.sync_copy(deg_v, degf_h.at[pl.ds(wid * SROWS, SROWS)])


def _encode_body(part_ref, degp_ref, w_ref, h_ref):
    agg = part_ref[0] + part_ref[1]
    deg = jnp.maximum(jnp.sum(degp_ref[...], axis=0), 1.0)
    h = jnp.dot(agg / deg[:, None], w_ref[...],
                preferred_element_type=jnp.float32)
    h_ref[...] = jnp.maximum(h, 0.0)


def _predict_body(h_h, aif_h, bif_h, pred_h,
                  ai_c0, ai_c1, bi_c0, bi_c1, ar0, ar1, br0, br1, ob,
                  sem_i, sem_g):
    cid = lax.axis_index("c")
    sid = lax.axis_index("s")
    wid = cid * NS + sid
    lane_iota = lax.iota(jnp.int32, 16)
    ai_c = (ai_c0, ai_c1)
    bi_c = (bi_c0, bi_c1)
    ar = (ar0, ar1)
    br = (br0, br1)

    def base(j):
        return wid * PPW + j * C

    def drain_idx():
        pltpu.make_async_copy(aif_h.at[pl.ds(0, C)], ai_c0, sem_i).wait()
        pltpu.make_async_copy(aif_h.at[pl.ds(0, C)], bi_c0, sem_i).wait()

    def fire_idx(j, k):
        pltpu.async_copy(aif_h.at[pl.ds(base(j), C)], ai_c[k], sem_i)
        pltpu.async_copy(bif_h.at[pl.ds(base(j), C)], bi_c[k], sem_i)

    def drain_rows(k):
        pltpu.make_async_copy(h_h.at[pl.ds(0, C)], ar[k], sem_g).wait()
        pltpu.make_async_copy(h_h.at[pl.ds(0, C)], br[k], sem_g).wait()

    def fire_rows(k):
        pltpu.async_copy(h_h.at[ai_c[k]], ar[k], sem_g)
        pltpu.async_copy(h_h.at[bi_c[k]], br[k], sem_g)

    fire_idx(0, 0)
    fire_idx(1, 1)
    drain_idx()
    fire_rows(0)

    @pl.loop(0, PCW, step=2)
    def _(jj):
        for b in range(2):
            j = jj + b
            k, o = b, 1 - b
            drain_rows(k)
            drain_idx()
            fire_rows(o)
            fire_idx(j + 2, k)

            def ebody(e, out16):
                a_e = ar[k].at[e]
                b_e = br[k].at[e]
                acc = a_e[pl.ds(0, 16)] * b_e[pl.ds(0, 16)]
                for t in range(1, 8):
                    acc = acc + (a_e[pl.ds(t * 16, 16)]
                                 * b_e[pl.ds(t * 16, 16)])
                s = jnp.sum(acc)
                lane = e % 16
                out16 = jnp.where(lane_iota == lane, s, out16)

                @pl.when(lane == 15)
                def _store():
                    ob[pl.ds(j * C + e - 15, 16)] = out16

                return out16

            lax.fori_loop(0, C, ebody, jnp.zeros((16,), jnp.float32))

    drain_rows(0)
    drain_idx()
    pltpu.sync_copy(ob, pred_h.at[pl.ds(wid * PPW, PPW)])


def kernel(x, edge_index, edge_label_index, W):
    x = x.astype(jnp.float32)
    src = edge_index[0].astype(jnp.int32)
    dst = edge_index[1].astype(jnp.int32)
    pad = EPAD - E
    srcf = jnp.concatenate([src, jnp.zeros((pad,), jnp.int32)])
    dstf = jnp.concatenate([dst, jnp.full((pad,), SROWS - 1, jnp.int32)])
    zrows = jnp.zeros((RPT, D), jnp.float32)
    zdeg = jnp.zeros((SROWS,), jnp.float32)
    lpad = jnp.zeros((PPAD - P,), jnp.int32)
    aif = jnp.concatenate([edge_label_index[0].astype(jnp.int32), lpad])
    bif = jnp.concatenate([edge_label_index[1].astype(jnp.int32), lpad])

    mesh = plsc.VectorSubcoreMesh(core_axis_name="c", subcore_axis_name="s")
    sc_params = pltpu.CompilerParams(needs_layout_passes=False)

    aggregate = functools.partial(
        pl.kernel,
        out_type=(
            jax.ShapeDtypeStruct((NC, SROWS, D), jnp.float32),
            jax.ShapeDtypeStruct((NW * SROWS,), jnp.float32),
        ),
        mesh=mesh,
        scratch_types=[
            pltpu.VMEM((C,), jnp.int32),
            pltpu.VMEM((C,), jnp.int32),
            pltpu.VMEM((C,), jnp.int32),
            pltpu.VMEM((C,), jnp.int32),
            pltpu.VMEM((C, D), jnp.float32),
            pltpu.VMEM((C, D), jnp.float32),
            pltpu.VMEM((SROWS,), jnp.float32),
            pltpu.VMEM((32,), jnp.int32),
            pltpu.VMEM_SHARED((SROWS, D), jnp.float32),
            pltpu.SemaphoreType.DMA,
            pltpu.SemaphoreType.DMA,
        ],
        compiler_params=sc_params,
    )(_aggregate_body)
    part, degf = aggregate(x, srcf, dstf, zrows, zdeg)
    degp = degf.reshape(NW, SROWS)

    h = pl.pallas_call(
        _encode_body,
        grid=(SROWS // BR,),
        in_specs=[
            pl.BlockSpec((NC, BR, D), lambda i: (0, i, 0)),
            pl.BlockSpec((NW, BR), lambda i: (0, i)),
            pl.BlockSpec((D, D), lambda i: (0, 0)),
        ],
        out_specs=pl.BlockSpec((BR, D), lambda i: (i, 0)),
        out_shape=jax.ShapeDtypeStruct((SROWS, D), jnp.float32),
    )(part, degp, W.astype(jnp.float32))

    predict = functools.partial(
        pl.kernel,
        out_type=jax.ShapeDtypeStruct((P,), jnp.float32),
        mesh=mesh,
        scratch_types=[
            pltpu.VMEM((C,), jnp.int32),
            pltpu.VMEM((C,), jnp.int32),
            pltpu.VMEM((C,), jnp.int32),
            pltpu.VMEM((C,), jnp.int32),
            pltpu.VMEM((C, D), jnp.float32),
            pltpu.VMEM((C, D), jnp.float32),
            pltpu.VMEM((C, D), jnp.float32),
            pltpu.VMEM((C, D), jnp.float32),
            pltpu.VMEM((PPW,), jnp.float32),
            pltpu.SemaphoreType.DMA,
            pltpu.SemaphoreType.DMA,
        ],
        compiler_params=sc_params,
    )(_predict_body)
    return predict(h, aif, bif)
